# baseline (speedup 1.0000x reference)
import jax
import jax.numpy as jnp
from jax.experimental import pallas as pl
from jax.experimental.pallas import tpu as pltpu

_LRELU = 0.01


def _lrelu(v):
    return jnp.where(v >= 0, v, _LRELU * v)


def _vq_kernel(zt_ref, cb_ref, out_ref):
    zt = zt_ref[...]
    cb = cb_ref[...]
    s = jax.lax.dot(cb, zt, preferred_element_type=jnp.float32)
    cbsq = jnp.sum(cb * cb, axis=1, keepdims=True)
    d = cbsq - 2.0 * s
    idx = jnp.argmin(d, axis=0)
    oh = (jax.lax.broadcasted_iota(jnp.int32, d.shape, 0)
          == idx[None, :]).astype(jnp.float32)
    out_ref[...] = jax.lax.dot_general(
        cb, oh, (((0,), (0,)), ((), ())),
        preferred_element_type=jnp.float32)


def _quantize(z, cb):
    B, C, D, H, W = z.shape
    M = B * D * H * W
    zt = jnp.transpose(z, (1, 0, 2, 3, 4)).reshape(C, M)
    Mp = max(128, M)
    if Mp % 128:
        Mp += 128 - Mp % 128
    if Mp != M:
        zt = jnp.pad(zt, ((0, 0), (0, Mp - M)))
    bm = min(Mp, 2048)
    grid = Mp // bm
    q = pl.pallas_call(
        _vq_kernel,
        grid=(grid,),
        in_specs=[
            pl.BlockSpec((C, bm), lambda i: (0, i)),
            pl.BlockSpec(cb.shape, lambda i: (0, 0)),
        ],
        out_specs=pl.BlockSpec((C, bm), lambda i: (0, i)),
        out_shape=jax.ShapeDtypeStruct((C, Mp), jnp.float32),
    )(zt, cb)
    q = q[:, :M].reshape(C, B, D, H, W)
    return jnp.transpose(q, (1, 0, 2, 3, 4))


def _make_block_kernel(kind, Ci, Co, D, H, W):
    n1 = Co * Ci * 27

    def kern(*refs):
        if kind == 'tail':
            y1_ref, ysk_ref, w_ref, sc_ref, out_ref, s1, s2, s3 = refs
            w2_base = 0
        else:
            x_ref, w_ref, sc_ref, out_ref, s1, s2, s3 = refs
            w2_base = 2 * n1
        b1b, b2a, b2b, scale = sc_ref[0], sc_ref[1], sc_ref[2], sc_ref[3]
        s1[...] = jnp.zeros(s1.shape, jnp.float32)
        s2[...] = jnp.zeros(s2.shape, jnp.float32)
        s3[...] = jnp.zeros(s3.shape, jnp.float32)

        if kind == 'tail':
            def loop1(f, carry):
                for co in range(Co):
                    s1[co, pl.ds(f + 1, 1), 1:H + 1, 1:W + 1] = _lrelu(
                        y1_ref[0, co, pl.ds(f, 1), :, :] + b1b)
                    s3[co, pl.ds(f + 1, 1), 1:H + 1, 1:W + 1] = \
                        ysk_ref[0, co, pl.ds(f, 1), :, :]
                return carry
        else:
            def src(ci, kd, kh, kw, f):
                return x_ref[0, ci, pl.ds(f + kd, 1), kh:kh + H, kw:kw + W]

            def loop1(f, carry):
                acc1 = [None] * Co
                acc2 = [None] * Co
                for ci in range(Ci):
                    for kd in range(3):
                        for kh in range(3):
                            for kw in range(3):
                                v = src(ci, kd, kh, kw, f)
                                for co in range(Co):
                                    n = (((co * Ci + ci) * 3 + kd) * 3 + kh) * 3 + kw
                                    t1 = w_ref[n] * v
                                    t2 = w_ref[n1 + n] * v
                                    acc1[co] = t1 if acc1[co] is None else acc1[co] + t1
                                    acc2[co] = t2 if acc2[co] is None else acc2[co] + t2
                for co in range(Co):
                    s1[co, pl.ds(f + 1, 1), 1:H + 1, 1:W + 1] = _lrelu(acc1[co] + b1b)
                    s3[co, pl.ds(f + 1, 1), 1:H + 1, 1:W + 1] = acc2[co]
                return carry

        jax.lax.fori_loop(0, D, loop1, 0)

        def _nca_row(s_ref, co, e):
            acc = None
            for a in (0, 1):
                for b in (0, 1):
                    for c in (0, 1):
                        t = s_ref[co, pl.ds(e + 1 - a, 1),
                                  1 - b:1 - b + H, 1 - c:1 - c + W]
                        acc = t if acc is None else acc + t
            return acc * 0.125

        def loop2(e, carry):
            for co in range(Co):
                s2[co, pl.ds(e + 1, 1), 1:H + 1, 1:W + 1] = (
                    _nca_row(s1, co, e) + b2a)
            return carry

        jax.lax.fori_loop(0, D, loop2, 0)

        def loop3(d, carry):
            accs = [None] * Co
            for ci in range(Co):
                for kd in range(3):
                    for kh in range(3):
                        for kw in range(3):
                            v = s2[ci, pl.ds(d + kd, 1), kh:kh + H, kw:kw + W]
                            for co in range(Co):
                                n = (w2_base
                                     + (((co * Co + ci) * 3 + kd) * 3 + kh) * 3 + kw)
                                t = w_ref[n] * v
                                accs[co] = t if accs[co] is None else accs[co] + t
            for co in range(Co):
                out_ref[0, co, pl.ds(d, 1), :, :] = _lrelu(
                    accs[co] * scale + b2b + _nca_row(s3, co, d))
            return carry

        jax.lax.fori_loop(0, D, loop3, 0)

    return kern


def _block_call(p, x_in, kind, Ci, Co, D, H, W, ysk=None):
    sc = jnp.concatenate([p['b1b'], p['b2a'], p['b2b'], p['scale']])
    if kind == 'level':
        B = x_in.shape[0]
        w = jnp.concatenate([p['w1'].reshape(-1), p['wskip'].reshape(-1),
                             p['w2'].reshape(-1)])
        xin = jnp.pad(x_in, ((0, 0), (0, 0), (1, 1), (1, 1), (1, 1)))
        ins = [(xin, (1, Ci, D + 2, H + 2, W + 2))]
    else:
        B = x_in.shape[0]
        w = p['w2'].reshape(-1)
        ins = [(x_in, (1, Co, D, H, W)), (ysk, (1, Co, D, H, W))]

    in_specs = [pl.BlockSpec(shp, lambda b, n=len(shp) - 1: (b,) + (0,) * n)
                for _, shp in ins]
    in_specs += [
        pl.BlockSpec(w.shape, lambda b: (0,), memory_space=pltpu.SMEM),
        pl.BlockSpec(sc.shape, lambda b: (0,), memory_space=pltpu.SMEM),
    ]
    return pl.pallas_call(
        _make_block_kernel(kind, Ci, Co, D, H, W),
        grid=(B,),
        in_specs=in_specs,
        out_specs=pl.BlockSpec((1, Co, D, H, W), lambda b: (b, 0, 0, 0, 0)),
        out_shape=jax.ShapeDtypeStruct((B, Co, D, H, W), jnp.float32),
        scratch_shapes=[
            pltpu.VMEM((Co, D + 2, H + 2, W + 2), jnp.float32),
            pltpu.VMEM((Co, D + 2, H + 2, W + 2), jnp.float32),
            pltpu.VMEM((Co, D + 2, H + 2, W + 2), jnp.float32),
        ],
    )(*[t for t, _ in ins], w, sc)


def _fixup_tail_pallas(p, x, kind_head, Co, D, H, W):
    xb = x + p['b1a']
    if kind_head == 'up':
        y1 = _convT3d(xb, p['w1'])
        ysk = _convT3d(xb, p['wskip'])
    else:
        y1 = _conv3d(xb, p['w1'], 2, 1)
        ysk = _conv3d(xb, p['wskip'], 2, 1)
    return _block_call(p, y1, 'tail', Co, Co, D, H, W, ysk=ysk)


def _d1_kernel(D, H, W):
    def kern(x_ref, w_ref, sc_ref, out_ref, sy):
        sy[...] = jnp.zeros(sy.shape, jnp.float32)

        def loop1(f, carry):
            accs = [None] * 8
            for ci in range(8):
                for kd in range(3):
                    for kh in range(3):
                        for kw in range(3):
                            v = x_ref[0, ci, pl.ds(f + kd, 1),
                                      kh:kh + H, kw:kw + W]
                            for co in range(8):
                                n = (((co * 8 + ci) * 3 + kd) * 3 + kh) * 3 + kw
                                t = w_ref[n] * v
                                accs[co] = t if accs[co] is None else accs[co] + t
            for co in range(8):
                sy[co, pl.ds(f + 1, 1), 1:H + 1, 1:W + 1] = (
                    accs[co] + sc_ref[co])
            return carry

        jax.lax.fori_loop(0, D, loop1, 0)

        def loop2(i, carry):
            for a in (0, 1):
                for b in (0, 1):
                    for c in (0, 1):
                        acc = None
                        for dd in (0, 1):
                            for dh in (0, 1):
                                for dw in (0, 1):
                                    ch = (4 * ((a - dd) & 1)
                                          + 2 * ((b - dh) & 1) + ((c - dw) & 1))
                                    t = sy[ch,
                                           pl.ds(i + 1 - (1 if dd > a else 0), 1),
                                           1 - (1 if dh > b else 0):
                                           1 - (1 if dh > b else 0) + H,
                                           1 - (1 if dw > c else 0):
                                           1 - (1 if dw > c else 0) + W]
                                    acc = t if acc is None else acc + t
                        out_ref[0, 4 * a + 2 * b + c, pl.ds(i, 1), :, :] = \
                            acc * 0.125
            return carry

        jax.lax.fori_loop(0, D, loop2, 0)

    return kern


def _subpixel_pallas(p, x):
    B, C, D, H, W = x.shape
    xp = jnp.pad(x, ((0, 0), (0, 0), (1, 1), (1, 1), (1, 1)))
    w = p['w'].reshape(-1)
    bias = p['b']
    out = pl.pallas_call(
        _d1_kernel(D, H, W),
        grid=(B,),
        in_specs=[
            pl.BlockSpec((1, 8, D + 2, H + 2, W + 2),
                         lambda b: (b, 0, 0, 0, 0)),
            pl.BlockSpec(w.shape, lambda b: (0,), memory_space=pltpu.SMEM),
            pl.BlockSpec(bias.shape, lambda b: (0,), memory_space=pltpu.SMEM),
        ],
        out_specs=pl.BlockSpec((1, 8, D, H, W), lambda b: (b, 0, 0, 0, 0)),
        out_shape=jax.ShapeDtypeStruct((B, 8, D, H, W), jnp.float32),
        scratch_shapes=[
            pltpu.VMEM((8, D + 1, H + 1, W + 1), jnp.float32),
        ],
    )(xp, w, bias)
    v = out.reshape(B, 2, 2, 2, D, H, W)
    v = jnp.transpose(v, (0, 4, 1, 5, 2, 6, 3))
    return v.reshape(B, 1, 2 * D, 2 * H, 2 * W)


def _conv3d(x, w, stride=1, pad=1):
    return jax.lax.conv_general_dilated(x, w, (stride,) * 3, [(pad, pad)] * 3,
                                        dimension_numbers=('NCDHW', 'OIDHW', 'NCDHW'))


def _convT3d(x, w):
    wt = jnp.transpose(jnp.flip(w, axis=(2, 3, 4)), (1, 0, 2, 3, 4))
    return jax.lax.conv_general_dilated(x, wt, (1, 1, 1), [(2, 2), (2, 2), (2, 2)],
                                        lhs_dilation=(2, 2, 2),
                                        dimension_numbers=('NCDHW', 'OIDHW', 'NCDHW'))


def _nca(x):
    xp = jnp.pad(x, ((0, 0), (0, 0), (1, 0), (1, 0), (1, 0)))
    s = jax.lax.reduce_window(xp, 0.0, jax.lax.add, (1, 1, 2, 2, 2), (1, 1, 1, 1, 1), 'VALID')
    return s / 8.0


def _fixup(p, x, kind):
    if kind == 'up':
        c = _convT3d
    elif kind == 'down':
        c = lambda z, w: _conv3d(z, w, 2, 1)
    else:
        c = lambda z, w: _conv3d(z, w, 1, 1)
    out = c(x + p['b1a'], p['w1'])
    out = _nca(jax.nn.leaky_relu(out + p['b1b']))
    out = _conv3d(out + p['b2a'], p['w2'], 1, 1)
    out = out * p['scale'] + p['b2b']
    out = out + _nca(c(x + p['b1a'], p['wskip']))
    return jax.nn.leaky_relu(out)


def _subpixel(p, x):
    out = _conv3d(x, p['w'], 1, 1) + p['b'][None, :, None, None, None]
    B, C, D, H, W = out.shape
    c = C // 8
    v = out.reshape(B, c, 2, 2, 2, D, H, W)
    v = jnp.transpose(v, (0, 1, 5, 2, 6, 3, 7, 4)).reshape(B, c, 2 * D, 2 * H, 2 * W)
    return _nca(v)


def kernel(x, params):
    p = params
    e0 = _fixup(p['e0'], x, 'level')
    e1 = _fixup(p['e1'], e0, 'down')
    e2 = _fixup(p['e2'], e1, 'down')
    e3 = _fixup(p['e3'], e2, 'down')
    e4 = _fixup(p['e4'], e3, 'down')
    e5 = _fixup(p['e5'], e4, 'down')
    e6 = _fixup(p['e6'], e5, 'down')
    z2 = _conv3d(e2, p['pq2_w'], 1, 0)
    z4 = _conv3d(e4, p['pq4_w'], 1, 0)
    q2 = _quantize(z2, p['cb2'])
    q4 = _quantize(z4, p['cb4'])
    q6 = _quantize(e6, p['cb6'])
    d5 = _fixup(p['d5'], _fixup(p['d6'], q6, 'up'), 'up')
    d4 = _fixup(p['d4'], jnp.concatenate([d5, q4], 1), 'up')
    d3 = _fixup_tail_pallas(p['d3'], d4, 'up', 16, 16, 16, 16)
    d2 = _fixup_tail_pallas(p['d2'], jnp.concatenate([d3, q2], 1), 'up',
                            8, 32, 32, 32)
    return _subpixel_pallas(p['d1'], d2)


# speedup vs baseline: 1.0003x; 1.0003x over previous
import jax
import jax.numpy as jnp
from jax.experimental import pallas as pl
from jax.experimental.pallas import tpu as pltpu

_LRELU = 0.01


def _lrelu(v):
    return jnp.where(v >= 0, v, _LRELU * v)


def _vq_kernel(zt_ref, cb_ref, out_ref):
    zt = zt_ref[0]
    cb = cb_ref[...]
    s = jax.lax.dot(cb, zt, preferred_element_type=jnp.float32)
    cbsq = jnp.sum(cb * cb, axis=1, keepdims=True)
    d = cbsq - 2.0 * s
    idx = jnp.argmin(d, axis=0)
    oh = (jax.lax.broadcasted_iota(jnp.int32, d.shape, 0)
          == idx[None, :]).astype(jnp.float32)
    out_ref[0] = jax.lax.dot_general(
        cb, oh, (((0,), (0,)), ((), ())),
        preferred_element_type=jnp.float32)


def _quantize(z, cb):
    B, C, D, H, W = z.shape
    M = D * H * W
    zr = z.reshape(B, C, M)
    Mp = max(128, M)
    if Mp % 128:
        Mp += 128 - Mp % 128
    if Mp != M:
        zr = jnp.pad(zr, ((0, 0), (0, 0), (0, Mp - M)))
    bm = min(Mp, 2048)
    q = pl.pallas_call(
        _vq_kernel,
        grid=(B, Mp // bm),
        in_specs=[
            pl.BlockSpec((1, C, bm), lambda b, i: (b, 0, i)),
            pl.BlockSpec(cb.shape, lambda b, i: (0, 0)),
        ],
        out_specs=pl.BlockSpec((1, C, bm), lambda b, i: (b, 0, i)),
        out_shape=jax.ShapeDtypeStruct((B, C, Mp), jnp.float32),
    )(zr, cb)
    return q[:, :, :M].reshape(B, C, D, H, W)


def _make_block_kernel(kind, Ci, Co, D, H, W):
    n1 = Co * Ci * 27

    def kern(*refs):
        if kind == 'tail':
            y1_ref, ysk_ref, w_ref, sc_ref, out_ref, s1, s2, s3 = refs
            w2_base = 0
        else:
            x_ref, w_ref, sc_ref, out_ref, s1, s2, s3 = refs
            w2_base = 2 * n1
        b1b, b2a, b2b, scale = sc_ref[0], sc_ref[1], sc_ref[2], sc_ref[3]
        s1[...] = jnp.zeros(s1.shape, jnp.float32)
        s2[...] = jnp.zeros(s2.shape, jnp.float32)
        s3[...] = jnp.zeros(s3.shape, jnp.float32)

        if kind == 'tail':
            def loop1(f, carry):
                for co in range(Co):
                    s1[co, pl.ds(f + 1, 1), 1:H + 1, 1:W + 1] = _lrelu(
                        y1_ref[0, co, pl.ds(f, 1), :, :] + b1b)
                    s3[co, pl.ds(f + 1, 1), 1:H + 1, 1:W + 1] = \
                        ysk_ref[0, co, pl.ds(f, 1), :, :]
                return carry
        else:
            def src(ci, kd, kh, kw, f):
                return x_ref[0, ci, pl.ds(f + kd, 1), kh:kh + H, kw:kw + W]

            def loop1(f, carry):
                acc1 = [None] * Co
                acc2 = [None] * Co
                for ci in range(Ci):
                    for kd in range(3):
                        for kh in range(3):
                            for kw in range(3):
                                v = src(ci, kd, kh, kw, f)
                                for co in range(Co):
                                    n = (((co * Ci + ci) * 3 + kd) * 3 + kh) * 3 + kw
                                    t1 = w_ref[n] * v
                                    t2 = w_ref[n1 + n] * v
                                    acc1[co] = t1 if acc1[co] is None else acc1[co] + t1
                                    acc2[co] = t2 if acc2[co] is None else acc2[co] + t2
                for co in range(Co):
                    s1[co, pl.ds(f + 1, 1), 1:H + 1, 1:W + 1] = _lrelu(acc1[co] + b1b)
                    s3[co, pl.ds(f + 1, 1), 1:H + 1, 1:W + 1] = acc2[co]
                return carry

        jax.lax.fori_loop(0, D, loop1, 0)

        def _nca_row(s_ref, co, e):
            acc = None
            for a in (0, 1):
                for b in (0, 1):
                    for c in (0, 1):
                        t = s_ref[co, pl.ds(e + 1 - a, 1),
                                  1 - b:1 - b + H, 1 - c:1 - c + W]
                        acc = t if acc is None else acc + t
            return acc * 0.125

        def loop2(e, carry):
            for co in range(Co):
                s2[co, pl.ds(e + 1, 1), 1:H + 1, 1:W + 1] = (
                    _nca_row(s1, co, e) + b2a)
            return carry

        jax.lax.fori_loop(0, D, loop2, 0)

        def loop3(d, carry):
            accs = [None] * Co
            for ci in range(Co):
                for kd in range(3):
                    for kh in range(3):
                        for kw in range(3):
                            v = s2[ci, pl.ds(d + kd, 1), kh:kh + H, kw:kw + W]
                            for co in range(Co):
                                n = (w2_base
                                     + (((co * Co + ci) * 3 + kd) * 3 + kh) * 3 + kw)
                                t = w_ref[n] * v
                                accs[co] = t if accs[co] is None else accs[co] + t
            for co in range(Co):
                out_ref[0, co, pl.ds(d, 1), :, :] = _lrelu(
                    accs[co] * scale + b2b + _nca_row(s3, co, d))
            return carry

        jax.lax.fori_loop(0, D, loop3, 0)

    return kern


def _block_call(p, x_in, kind, Ci, Co, D, H, W, ysk=None):
    sc = jnp.concatenate([p['b1b'], p['b2a'], p['b2b'], p['scale']])
    if kind == 'level':
        B = x_in.shape[0]
        w = jnp.concatenate([p['w1'].reshape(-1), p['wskip'].reshape(-1),
                             p['w2'].reshape(-1)])
        xin = jnp.pad(x_in, ((0, 0), (0, 0), (1, 1), (1, 1), (1, 1)))
        ins = [(xin, (1, Ci, D + 2, H + 2, W + 2))]
    else:
        B = x_in.shape[0]
        w = p['w2'].reshape(-1)
        ins = [(x_in, (1, Co, D, H, W)), (ysk, (1, Co, D, H, W))]

    in_specs = [pl.BlockSpec(shp, lambda b, n=len(shp) - 1: (b,) + (0,) * n)
                for _, shp in ins]
    in_specs += [
        pl.BlockSpec(w.shape, lambda b: (0,), memory_space=pltpu.SMEM),
        pl.BlockSpec(sc.shape, lambda b: (0,), memory_space=pltpu.SMEM),
    ]
    return pl.pallas_call(
        _make_block_kernel(kind, Ci, Co, D, H, W),
        grid=(B,),
        in_specs=in_specs,
        out_specs=pl.BlockSpec((1, Co, D, H, W), lambda b: (b, 0, 0, 0, 0)),
        out_shape=jax.ShapeDtypeStruct((B, Co, D, H, W), jnp.float32),
        scratch_shapes=[
            pltpu.VMEM((Co, D + 2, H + 2, W + 2), jnp.float32),
            pltpu.VMEM((Co, D + 2, H + 2, W + 2), jnp.float32),
            pltpu.VMEM((Co, D + 2, H + 2, W + 2), jnp.float32),
        ],
    )(*[t for t, _ in ins], w, sc)


def _fixup_tail_pallas(p, x, kind_head, Co, D, H, W):
    xb = x + p['b1a']
    if kind_head == 'up':
        y1 = _convT3d(xb, p['w1'])
        ysk = _convT3d(xb, p['wskip'])
    else:
        y1 = _conv3d(xb, p['w1'], 2, 1)
        ysk = _conv3d(xb, p['wskip'], 2, 1)
    return _block_call(p, y1, 'tail', Co, Co, D, H, W, ysk=ysk)


def _d1_kernel(D, H, W):
    def kern(x_ref, w_ref, sc_ref, out_ref, sy):
        sy[...] = jnp.zeros(sy.shape, jnp.float32)

        def loop1(f, carry):
            accs = [None] * 8
            for ci in range(8):
                for kd in range(3):
                    for kh in range(3):
                        for kw in range(3):
                            v = x_ref[0, ci, pl.ds(f + kd, 1),
                                      kh:kh + H, kw:kw + W]
                            for co in range(8):
                                n = (((co * 8 + ci) * 3 + kd) * 3 + kh) * 3 + kw
                                t = w_ref[n] * v
                                accs[co] = t if accs[co] is None else accs[co] + t
            for co in range(8):
                sy[co, pl.ds(f + 1, 1), 1:H + 1, 1:W + 1] = (
                    accs[co] + sc_ref[co])
            return carry

        jax.lax.fori_loop(0, D, loop1, 0)

        def loop2(i, carry):
            for a in (0, 1):
                for b in (0, 1):
                    for c in (0, 1):
                        acc = None
                        for dd in (0, 1):
                            for dh in (0, 1):
                                for dw in (0, 1):
                                    ch = (4 * ((a - dd) & 1)
                                          + 2 * ((b - dh) & 1) + ((c - dw) & 1))
                                    t = sy[ch,
                                           pl.ds(i + 1 - (1 if dd > a else 0), 1),
                                           1 - (1 if dh > b else 0):
                                           1 - (1 if dh > b else 0) + H,
                                           1 - (1 if dw > c else 0):
                                           1 - (1 if dw > c else 0) + W]
                                    acc = t if acc is None else acc + t
                        out_ref[0, 4 * a + 2 * b + c, pl.ds(i, 1), :, :] = \
                            acc * 0.125
            return carry

        jax.lax.fori_loop(0, D, loop2, 0)

    return kern


def _subpixel_pallas(p, x):
    B, C, D, H, W = x.shape
    xp = jnp.pad(x, ((0, 0), (0, 0), (1, 1), (1, 1), (1, 1)))
    w = p['w'].reshape(-1)
    bias = p['b']
    out = pl.pallas_call(
        _d1_kernel(D, H, W),
        grid=(B,),
        in_specs=[
            pl.BlockSpec((1, 8, D + 2, H + 2, W + 2),
                         lambda b: (b, 0, 0, 0, 0)),
            pl.BlockSpec(w.shape, lambda b: (0,), memory_space=pltpu.SMEM),
            pl.BlockSpec(bias.shape, lambda b: (0,), memory_space=pltpu.SMEM),
        ],
        out_specs=pl.BlockSpec((1, 8, D, H, W), lambda b: (b, 0, 0, 0, 0)),
        out_shape=jax.ShapeDtypeStruct((B, 8, D, H, W), jnp.float32),
        scratch_shapes=[
            pltpu.VMEM((8, D + 1, H + 1, W + 1), jnp.float32),
        ],
    )(xp, w, bias)
    v = out.reshape(B, 2, 2, 2, D, H, W)
    v = jnp.transpose(v, (0, 4, 1, 5, 2, 6, 3))
    return v.reshape(B, 1, 2 * D, 2 * H, 2 * W)


def _conv3d(x, w, stride=1, pad=1):
    return jax.lax.conv_general_dilated(x, w, (stride,) * 3, [(pad, pad)] * 3,
                                        dimension_numbers=('NCDHW', 'OIDHW', 'NCDHW'))


def _convT3d(x, w):
    wt = jnp.transpose(jnp.flip(w, axis=(2, 3, 4)), (1, 0, 2, 3, 4))
    return jax.lax.conv_general_dilated(x, wt, (1, 1, 1), [(2, 2), (2, 2), (2, 2)],
                                        lhs_dilation=(2, 2, 2),
                                        dimension_numbers=('NCDHW', 'OIDHW', 'NCDHW'))


def _nca(x):
    xp = jnp.pad(x, ((0, 0), (0, 0), (1, 0), (1, 0), (1, 0)))
    s = jax.lax.reduce_window(xp, 0.0, jax.lax.add, (1, 1, 2, 2, 2), (1, 1, 1, 1, 1), 'VALID')
    return s / 8.0


def _fixup(p, x, kind):
    if kind == 'up':
        c = _convT3d
    elif kind == 'down':
        c = lambda z, w: _conv3d(z, w, 2, 1)
    else:
        c = lambda z, w: _conv3d(z, w, 1, 1)
    out = c(x + p['b1a'], p['w1'])
    out = _nca(jax.nn.leaky_relu(out + p['b1b']))
    out = _conv3d(out + p['b2a'], p['w2'], 1, 1)
    out = out * p['scale'] + p['b2b']
    out = out + _nca(c(x + p['b1a'], p['wskip']))
    return jax.nn.leaky_relu(out)


def _subpixel(p, x):
    out = _conv3d(x, p['w'], 1, 1) + p['b'][None, :, None, None, None]
    B, C, D, H, W = out.shape
    c = C // 8
    v = out.reshape(B, c, 2, 2, 2, D, H, W)
    v = jnp.transpose(v, (0, 1, 5, 2, 6, 3, 7, 4)).reshape(B, c, 2 * D, 2 * H, 2 * W)
    return _nca(v)


def kernel(x, params):
    p = params
    e0 = _fixup(p['e0'], x, 'level')
    e1 = _fixup(p['e1'], e0, 'down')
    e2 = _fixup(p['e2'], e1, 'down')
    e3 = _fixup(p['e3'], e2, 'down')
    e4 = _fixup(p['e4'], e3, 'down')
    e5 = _fixup(p['e5'], e4, 'down')
    e6 = _fixup(p['e6'], e5, 'down')
    z2 = _conv3d(e2, p['pq2_w'], 1, 0)
    z4 = _conv3d(e4, p['pq4_w'], 1, 0)
    q2 = _quantize(z2, p['cb2'])
    q4 = _quantize(z4, p['cb4'])
    q6 = _quantize(e6, p['cb6'])
    d5 = _fixup(p['d5'], _fixup(p['d6'], q6, 'up'), 'up')
    d4 = _fixup(p['d4'], jnp.concatenate([d5, q4], 1), 'up')
    d3 = _fixup_tail_pallas(p['d3'], d4, 'up', 16, 16, 16, 16)
    d2 = _fixup_tail_pallas(p['d2'], jnp.concatenate([d3, q2], 1), 'up',
                            8, 32, 32, 32)
    return _subpixel_pallas(p['d1'], d2)


# speedup vs baseline: 1.0360x; 1.0358x over previous
import jax
import jax.numpy as jnp
from jax.experimental import pallas as pl
from jax.experimental.pallas import tpu as pltpu

_LRELU = 0.01


def _lrelu(v):
    return jnp.where(v >= 0, v, _LRELU * v)


def _vq_kernel(zt_ref, cb_ref, out_ref):
    zt = zt_ref[0]
    cb = cb_ref[...]
    s = jax.lax.dot(cb, zt, preferred_element_type=jnp.float32)
    cbsq = jnp.sum(cb * cb, axis=1, keepdims=True)
    d = cbsq - 2.0 * s
    idx = jnp.argmin(d, axis=0)
    oh = (jax.lax.broadcasted_iota(jnp.int32, d.shape, 0)
          == idx[None, :]).astype(jnp.float32)
    out_ref[0] = jax.lax.dot_general(
        cb, oh, (((0,), (0,)), ((), ())),
        preferred_element_type=jnp.float32)


def _quantize(z, cb):
    B, C, D, H, W = z.shape
    M = D * H * W
    zr = z.reshape(B, C, M)
    Mp = max(128, M)
    if Mp % 128:
        Mp += 128 - Mp % 128
    if Mp != M:
        zr = jnp.pad(zr, ((0, 0), (0, 0), (0, Mp - M)))
    bm = min(Mp, 2048)
    q = pl.pallas_call(
        _vq_kernel,
        grid=(B, Mp // bm),
        in_specs=[
            pl.BlockSpec((1, C, bm), lambda b, i: (b, 0, i)),
            pl.BlockSpec(cb.shape, lambda b, i: (0, 0)),
        ],
        out_specs=pl.BlockSpec((1, C, bm), lambda b, i: (b, 0, i)),
        out_shape=jax.ShapeDtypeStruct((B, C, Mp), jnp.float32),
    )(zr, cb)
    return q[:, :, :M].reshape(B, C, D, H, W)


def _make_block_kernel(kind, Ci, Co, D, H, W):
    n1 = Co * Ci * 27

    def kern(*refs):
        if kind == 'tail':
            y1_ref, ysk_ref, w_ref, sc_ref, out_ref, s1, s2, s3 = refs
            w2_base = 0
        else:
            x_ref, w_ref, sc_ref, out_ref, s1, s2, s3 = refs
            w2_base = 2 * n1
        b1b, b2a, b2b, scale = sc_ref[0], sc_ref[1], sc_ref[2], sc_ref[3]
        s1[...] = jnp.zeros(s1.shape, jnp.float32)
        s2[...] = jnp.zeros(s2.shape, jnp.float32)
        s3[...] = jnp.zeros(s3.shape, jnp.float32)

        if kind == 'tail':
            def loop1(f, carry):
                for co in range(Co):
                    s1[co, pl.ds(f + 1, 1), 1:H + 1, 1:W + 1] = _lrelu(
                        y1_ref[0, co, pl.ds(f, 1), :, :] + b1b)
                    s3[co, pl.ds(f + 1, 1), 1:H + 1, 1:W + 1] = \
                        ysk_ref[0, co, pl.ds(f, 1), :, :]
                return carry
        else:
            def src(ci, kd, kh, kw, f):
                return x_ref[0, ci, pl.ds(f + kd, 1), kh:kh + H, kw:kw + W]

            def loop1(f, carry):
                acc1 = [None] * Co
                acc2 = [None] * Co
                for ci in range(Ci):
                    for kd in range(3):
                        for kh in range(3):
                            for kw in range(3):
                                v = src(ci, kd, kh, kw, f)
                                for co in range(Co):
                                    n = (((co * Ci + ci) * 3 + kd) * 3 + kh) * 3 + kw
                                    t1 = w_ref[n] * v
                                    t2 = w_ref[n1 + n] * v
                                    acc1[co] = t1 if acc1[co] is None else acc1[co] + t1
                                    acc2[co] = t2 if acc2[co] is None else acc2[co] + t2
                for co in range(Co):
                    s1[co, pl.ds(f + 1, 1), 1:H + 1, 1:W + 1] = _lrelu(acc1[co] + b1b)
                    s3[co, pl.ds(f + 1, 1), 1:H + 1, 1:W + 1] = acc2[co]
                return carry

        jax.lax.fori_loop(0, D, loop1, 0)

        def _nca_row(s_ref, co, e):
            acc = None
            for a in (0, 1):
                for b in (0, 1):
                    for c in (0, 1):
                        t = s_ref[co, pl.ds(e + 1 - a, 1),
                                  1 - b:1 - b + H, 1 - c:1 - c + W]
                        acc = t if acc is None else acc + t
            return acc * 0.125

        def loop2(e, carry):
            for co in range(Co):
                s2[co, pl.ds(e + 1, 1), 1:H + 1, 1:W + 1] = (
                    _nca_row(s1, co, e) + b2a)
            return carry

        jax.lax.fori_loop(0, D, loop2, 0)

        def loop3(d, carry):
            accs = [None] * Co
            for ci in range(Co):
                for kd in range(3):
                    for kh in range(3):
                        for kw in range(3):
                            v = s2[ci, pl.ds(d + kd, 1), kh:kh + H, kw:kw + W]
                            for co in range(Co):
                                n = (w2_base
                                     + (((co * Co + ci) * 3 + kd) * 3 + kh) * 3 + kw)
                                t = w_ref[n] * v
                                accs[co] = t if accs[co] is None else accs[co] + t
            for co in range(Co):
                out_ref[0, co, pl.ds(d, 1), :, :] = _lrelu(
                    accs[co] * scale + b2b + _nca_row(s3, co, d))
            return carry

        jax.lax.fori_loop(0, D, loop3, 0)

    return kern


def _block_call(p, x_in, kind, Ci, Co, D, H, W, ysk=None):
    sc = jnp.concatenate([p['b1b'], p['b2a'], p['b2b'], p['scale']])
    if kind == 'level':
        B = x_in.shape[0]
        w = jnp.concatenate([p['w1'].reshape(-1), p['wskip'].reshape(-1),
                             p['w2'].reshape(-1)])
        xin = jnp.pad(x_in, ((0, 0), (0, 0), (1, 1), (1, 1), (1, 1)))
        ins = [(xin, (1, Ci, D + 2, H + 2, W + 2))]
    else:
        B = x_in.shape[0]
        w = p['w2'].reshape(-1)
        ins = [(x_in, (1, Co, D, H, W)), (ysk, (1, Co, D, H, W))]

    in_specs = [pl.BlockSpec(shp, lambda b, n=len(shp) - 1: (b,) + (0,) * n)
                for _, shp in ins]
    in_specs += [
        pl.BlockSpec(w.shape, lambda b: (0,), memory_space=pltpu.SMEM),
        pl.BlockSpec(sc.shape, lambda b: (0,), memory_space=pltpu.SMEM),
    ]
    return pl.pallas_call(
        _make_block_kernel(kind, Ci, Co, D, H, W),
        grid=(B,),
        in_specs=in_specs,
        out_specs=pl.BlockSpec((1, Co, D, H, W), lambda b: (b, 0, 0, 0, 0)),
        out_shape=jax.ShapeDtypeStruct((B, Co, D, H, W), jnp.float32),
        scratch_shapes=[
            pltpu.VMEM((Co, D + 2, H + 2, W + 2), jnp.float32),
            pltpu.VMEM((Co, D + 2, H + 2, W + 2), jnp.float32),
            pltpu.VMEM((Co, D + 2, H + 2, W + 2), jnp.float32),
        ],
    )(*[t for t, _ in ins], w, sc)


def _fixup_tail_pallas(p, x, kind_head, Co, D, H, W):
    xb = x + p['b1a']
    if kind_head == 'up':
        y1 = _convT3d(xb, p['w1'])
        ysk = _convT3d(xb, p['wskip'])
    else:
        y1 = _conv3d(xb, p['w1'], 2, 1)
        ysk = _conv3d(xb, p['wskip'], 2, 1)
    return _block_call(p, y1, 'tail', Co, Co, D, H, W, ysk=ysk)


def _d1_kernel(D, H, W):
    def kern(x_ref, w_ref, sc_ref, out_ref, sy):
        sy[...] = jnp.zeros(sy.shape, jnp.float32)

        def loop1(f, carry):
            accs = [None] * 8
            for ci in range(8):
                for kd in range(3):
                    for kh in range(3):
                        for kw in range(3):
                            v = x_ref[0, ci, pl.ds(f + kd, 1),
                                      kh:kh + H, kw:kw + W]
                            for co in range(8):
                                n = (((co * 8 + ci) * 3 + kd) * 3 + kh) * 3 + kw
                                t = w_ref[n] * v
                                accs[co] = t if accs[co] is None else accs[co] + t
            for co in range(8):
                sy[co, pl.ds(f + 1, 1), 1:H + 1, 1:W + 1] = (
                    accs[co] + sc_ref[co])
            return carry

        jax.lax.fori_loop(0, D, loop1, 0)

        def loop2(i, carry):
            for a in (0, 1):
                for b in (0, 1):
                    for c in (0, 1):
                        acc = None
                        for dd in (0, 1):
                            for dh in (0, 1):
                                for dw in (0, 1):
                                    ch = (4 * ((a - dd) & 1)
                                          + 2 * ((b - dh) & 1) + ((c - dw) & 1))
                                    t = sy[ch,
                                           pl.ds(i + 1 - (1 if dd > a else 0), 1),
                                           1 - (1 if dh > b else 0):
                                           1 - (1 if dh > b else 0) + H,
                                           1 - (1 if dw > c else 0):
                                           1 - (1 if dw > c else 0) + W]
                                    acc = t if acc is None else acc + t
                        out_ref[0, 4 * a + 2 * b + c, pl.ds(i, 1), :, :] = \
                            acc * 0.125
            return carry

        jax.lax.fori_loop(0, D, loop2, 0)

    return kern


def _subpixel_pallas(p, x):
    B, C, D, H, W = x.shape
    xp = jnp.pad(x, ((0, 0), (0, 0), (1, 1), (1, 1), (1, 1)))
    w = p['w'].reshape(-1)
    bias = p['b']
    out = pl.pallas_call(
        _d1_kernel(D, H, W),
        grid=(B,),
        in_specs=[
            pl.BlockSpec((1, 8, D + 2, H + 2, W + 2),
                         lambda b: (b, 0, 0, 0, 0)),
            pl.BlockSpec(w.shape, lambda b: (0,), memory_space=pltpu.SMEM),
            pl.BlockSpec(bias.shape, lambda b: (0,), memory_space=pltpu.SMEM),
        ],
        out_specs=pl.BlockSpec((1, 8, D, H, W), lambda b: (b, 0, 0, 0, 0)),
        out_shape=jax.ShapeDtypeStruct((B, 8, D, H, W), jnp.float32),
        scratch_shapes=[
            pltpu.VMEM((8, D + 1, H + 1, W + 1), jnp.float32),
        ],
    )(xp, w, bias)
    v = out.reshape(B, 2, 2, 2, D, H, W)
    v = jnp.transpose(v, (0, 4, 1, 5, 2, 6, 3))
    return v.reshape(B, 1, 2 * D, 2 * H, 2 * W)


def _conv3d(x, w, stride=1, pad=1):
    return jax.lax.conv_general_dilated(x, w, (stride,) * 3, [(pad, pad)] * 3,
                                        dimension_numbers=('NCDHW', 'OIDHW', 'NCDHW'))


def _convT3d(x, w):
    wt = jnp.transpose(jnp.flip(w, axis=(2, 3, 4)), (1, 0, 2, 3, 4))
    return jax.lax.conv_general_dilated(x, wt, (1, 1, 1), [(2, 2), (2, 2), (2, 2)],
                                        lhs_dilation=(2, 2, 2),
                                        dimension_numbers=('NCDHW', 'OIDHW', 'NCDHW'))


def _nca(x):
    xp = jnp.pad(x, ((0, 0), (0, 0), (1, 0), (1, 0), (1, 0)))
    s = jax.lax.reduce_window(xp, 0.0, jax.lax.add, (1, 1, 2, 2, 2), (1, 1, 1, 1, 1), 'VALID')
    return s / 8.0


def _fixup(p, x, kind):
    if kind == 'up':
        c = _convT3d
    elif kind == 'down':
        c = lambda z, w: _conv3d(z, w, 2, 1)
    else:
        c = lambda z, w: _conv3d(z, w, 1, 1)
    out = c(x + p['b1a'], p['w1'])
    out = _nca(jax.nn.leaky_relu(out + p['b1b']))
    out = _conv3d(out + p['b2a'], p['w2'], 1, 1)
    out = out * p['scale'] + p['b2b']
    out = out + _nca(c(x + p['b1a'], p['wskip']))
    return jax.nn.leaky_relu(out)


def _subpixel(p, x):
    out = _conv3d(x, p['w'], 1, 1) + p['b'][None, :, None, None, None]
    B, C, D, H, W = out.shape
    c = C // 8
    v = out.reshape(B, c, 2, 2, 2, D, H, W)
    v = jnp.transpose(v, (0, 1, 5, 2, 6, 3, 7, 4)).reshape(B, c, 2 * D, 2 * H, 2 * W)
    return _nca(v)


def kernel(x, params):
    p = params
    e0 = _fixup(p['e0'], x, 'level')
    e1 = _fixup(p['e1'], e0, 'down')
    e2 = _fixup(p['e2'], e1, 'down')
    e3 = _fixup(p['e3'], e2, 'down')
    e4 = _fixup(p['e4'], e3, 'down')
    e5 = _fixup(p['e5'], e4, 'down')
    e6 = _fixup(p['e6'], e5, 'down')
    z2 = _conv3d(e2, p['pq2_w'], 1, 0)
    z4 = _conv3d(e4, p['pq4_w'], 1, 0)
    q2 = _quantize(z2, p['cb2'])
    q4 = _quantize(z4, p['cb4'])
    q6 = _quantize(e6, p['cb6'])
    d5 = _fixup(p['d5'], _fixup(p['d6'], q6, 'up'), 'up')
    d4 = _fixup(p['d4'], jnp.concatenate([d5, q4], 1), 'up')
    d3 = _fixup_tail_pallas(p['d3'], d4, 'up', 16, 16, 16, 16)
    d2 = _fixup_tail_pallas(p['d2'], jnp.concatenate([d3, q2], 1), 'up',
                            8, 32, 32, 32)
    return _subpixel(p['d1'], d2)
